# baseline (speedup 1.0000x reference)
import functools

import jax
import jax.numpy as jnp
from jax import lax
from jax.experimental import pallas as pl
from jax.experimental.pallas import tpu as pltpu
from jax.experimental.pallas import tpu_sc as plsc

V = 1000000
D = 64
B = 16384
L = 16
NC, NS = 2, 16
NW = NC * NS
CH = 256
NFULL = 3906
MINI = NFULL * CH
MINIW = V - MINI
NLOC = 123
MINIOWN = 2
CAP = 48
RS = 32
TRASH = B
OUTR = B + 8
IDW = 2048

_mesh = plsc.VectorSubcoreMesh(core_axis_name="c", subcore_axis_name="s")


@functools.partial(
    pl.kernel,
    out_type=(
        jax.ShapeDtypeStruct((OUTR, 2 * D), jnp.float32),
        jax.ShapeDtypeStruct((OUTR, 2 * D), jnp.float32),
        jax.ShapeDtypeStruct((OUTR, 2 * D), jnp.float32),
    ),
    mesh=_mesh,
    compiler_params=pltpu.CompilerParams(needs_layout_passes=False),
    scratch_types=[
        pltpu.VMEM((2, D, CH), jnp.float32),
        pltpu.VMEM((2, D, CH), jnp.float32),
        pltpu.VMEM((D, MINIW), jnp.float32),
        pltpu.VMEM((NLOC * CAP,), jnp.int32),
        pltpu.VMEM((NLOC * CAP,), jnp.int32),
        pltpu.VMEM((NLOC * CAP,), jnp.int32),
        pltpu.VMEM((RS, 2 * D), jnp.float32),
        pltpu.VMEM((RS, 2 * D), jnp.float32),
        pltpu.VMEM((RS, 2 * D), jnp.float32),
        pltpu.VMEM((3, RS), jnp.int32),
        pltpu.VMEM((2, IDW), jnp.int32),
        pltpu.SMEM((3, NLOC), jnp.int32),
        pltpu.SemaphoreType.DMA,
        pltpu.SemaphoreType.DMA,
        pltpu.SemaphoreType.DMA,
        pltpu.SemaphoreType.DMA,
        pltpu.SemaphoreType.DMA,
        pltpu.SemaphoreType.DMA,
    ],
)
def _sweep(uids_h, pids_h, nids_h, utab_h, itab_h,
           urows_h, prows_h, nrows_h,
           ubuf, ibuf, minibuf, ubkt, pbkt, nbkt, rsu, rsp, rsn, slots, idst,
           cur, semu0, semu1, semi0, semi1, semid, semsc):
    w = lax.axis_index("s") * NC + lax.axis_index("c")
    lanes = lax.iota(jnp.int32, L)
    lane0 = lanes == 0
    trash = jnp.full((L,), TRASH, jnp.int32) + (w & 7)

    def zcur(i, c):
        cur[0, i] = 0
        cur[1, i] = 0
        cur[2, i] = 0
        return c

    lax.fori_loop(0, NLOC, zcur, 0)
    for li in range(3):
        for kk in range(RS // L):
            slots[li, pl.ds(kk * L, L)] = trash

    ids_list = (uids_h, pids_h, nids_h)
    bkts = (ubkt, pbkt, nbkt)
    nwin = B // IDW

    def fire_ids(li, wi):
        return pltpu.async_copy(
            ids_list[li].at[pl.ds(wi * IDW, IDW)], idst.at[(li * nwin + wi) % 2],
            semid)

    fire_ids(0, 0)
    for li in range(3):
        bkt = bkts[li]
        for wi in range(nwin):
            seq = li * nwin + wi
            bb = seq % 2
            if seq + 1 < 3 * nwin:
                nli, nwi = divmod(seq + 1, nwin)
                fire_ids(nli, nwi)
            pltpu.make_async_copy(
                ids_list[li].at[pl.ds(wi * IDW, IDW)], idst.at[bb], semid
            ).wait()

            def group(s, c):
                v = idst[bb, pl.ds(s * L, L)]
                g = lax.shift_right_logical(v, 8)
                own = (g & 31) == w
                o = v - g * CH
                slot = wi * IDW + s * L + lanes
                entry = slot * 1024 + o
                lvec = lax.shift_right_logical(g, 5)

                def cond(m):
                    return jnp.any(m)

                def body(m):
                    j16 = plsc.all_reduce_ffs(m)
                    sel = lanes == j16
                    e = jnp.max(jnp.where(sel, entry, 0))
                    l = jnp.max(jnp.where(sel, lvec, 0))
                    c0 = cur[li, l]

                    @pl.when(c0 < CAP)
                    def _():
                        plsc.store_scatter(
                            bkt, [jnp.full((L,), l * CAP + c0, jnp.int32)],
                            jnp.full((L,), e, jnp.int32), mask=lane0)
                        cur[li, l] = c0 + 1

                    return m & jnp.logical_not(sel)

                lax.while_loop(cond, body, own)
                return c

            lax.fori_loop(0, IDW // L, group, 0)

    cvecs = [lanes + 16 * k for k in range(D // L)]

    def chunk_start(j):
        return jnp.minimum(w + NW * j, NFULL - 1) * CH

    def fire_chunk(tab_h, buf, j, bb, sem):
        start = chunk_start(j)
        for h in range(4):
            pltpu.async_copy(
                tab_h.at[pl.ds(16 * h, 16), pl.ds(start, CH)],
                buf.at[bb, pl.ds(16 * h, 16)], sem)

    def wait_chunk(tab_h, buf, bb, sem):
        pltpu.make_async_copy(
            tab_h.at[:, pl.ds(0, CH)], buf.at[bb], sem).wait()

    def drain(bkt, li, j, cb, rstage, rows_h, rc, cnt):
        def one(m, rc):
            ev = bkt[pl.ds(j * CAP + (m & ~15), L)]
            sel = lanes == (m & 15)
            e = jnp.max(jnp.where(sel, ev, 0))
            o = e & 1023
            slot = lax.shift_right_logical(e, 10)
            r = rc & (RS - 1)
            ov = jnp.full((L,), o, jnp.int32)
            for k in range(D // L):
                vals = plsc.load_gather(cb, [cvecs[k], ov])
                rstage[r, pl.ds(k * L, L)] = vals
            plsc.store_scatter(
                slots, [jnp.full((L,), li, jnp.int32),
                        jnp.full((L,), r, jnp.int32)],
                jnp.full((L,), slot, jnp.int32), mask=lane0)
            rc = rc + 1

            @pl.when(rc & (RS - 1) == 0)
            def _():
                pltpu.async_copy(rstage, rows_h.at[slots.at[li]], semsc).wait()
                for kk in range(RS // L):
                    slots[li, pl.ds(kk * L, L)] = trash

            return rc

        return lax.fori_loop(0, cnt, one, rc)

    def full_cnt(li, j):
        c = cur[li, j]
        return jnp.where((j == NLOC - 1) & (w == MINIOWN), 0, c)

    def mini_cnt(li):
        return jnp.where(w == MINIOWN, cur[li, NLOC - 1], 0)

    def drains(j, ub_bb, ib_bb, rcu, rcp, rcn):
        wait_chunk(utab_h, ubuf, ub_bb, semu0 if ub_bb == 0 else semu1)
        rcu = drain(ubkt, 0, j, ubuf.at[ub_bb], rsu, urows_h, rcu,
                    full_cnt(0, j))
        wait_chunk(itab_h, ibuf, ib_bb, semi0 if ib_bb == 0 else semi1)
        rcp = drain(pbkt, 1, j, ibuf.at[ib_bb], rsp, prows_h, rcp,
                    full_cnt(1, j))
        rcn = drain(nbkt, 2, j, ibuf.at[ib_bb], rsn, nrows_h, rcn,
                    full_cnt(2, j))
        return rcu, rcp, rcn

    rcu = jnp.int32(0)
    rcp = jnp.int32(0)
    rcn = jnp.int32(0)

    fire_chunk(utab_h, ubuf, jnp.int32(0), 0, semu0)
    fire_chunk(itab_h, ibuf, jnp.int32(0), 0, semi0)

    def sweep(jj, carry):
        rcu, rcp, rcn = carry
        j0 = 2 * jj
        fire_chunk(utab_h, ubuf, j0 + 1, 1, semu1)
        fire_chunk(itab_h, ibuf, j0 + 1, 1, semi1)
        rcu, rcp, rcn = drains(j0, 0, 0, rcu, rcp, rcn)

        @pl.when(j0 + 2 < NLOC)
        def _():
            fire_chunk(utab_h, ubuf, j0 + 2, 0, semu0)
            fire_chunk(itab_h, ibuf, j0 + 2, 0, semi0)

        rcu, rcp, rcn = drains(j0 + 1, 1, 1, rcu, rcp, rcn)
        return rcu, rcp, rcn

    rcu, rcp, rcn = lax.fori_loop(0, NLOC // 2, sweep, (rcu, rcp, rcn))
    rcu, rcp, rcn = drains(NLOC - 1, 0, 0, rcu, rcp, rcn)

    pltpu.sync_copy(utab_h.at[:, pl.ds(MINI, MINIW)], minibuf)
    rcu = drain(ubkt, 0, NLOC - 1, minibuf, rsu, urows_h, rcu, mini_cnt(0))
    pltpu.sync_copy(itab_h.at[:, pl.ds(MINI, MINIW)], minibuf)
    rcp = drain(pbkt, 1, NLOC - 1, minibuf, rsp, prows_h, rcp, mini_cnt(1))
    rcn = drain(nbkt, 2, NLOC - 1, minibuf, rsn, nrows_h, rcn, mini_cnt(2))

    for li, rstage, rows_h, rc in ((0, rsu, urows_h, rcu),
                                   (1, rsp, prows_h, rcp),
                                   (2, rsn, nrows_h, rcn)):
        @pl.when(rc & (RS - 1) != 0)
        def _():
            pltpu.async_copy(rstage, rows_h.at[slots.at[li]], semsc).wait()


SB = B // NW
SCH = 128


@functools.partial(
    pl.kernel,
    out_type=(
        jax.ShapeDtypeStruct((B,), jnp.float32),
        jax.ShapeDtypeStruct((B,), jnp.float32),
    ),
    mesh=_mesh,
    compiler_params=pltpu.CompilerParams(needs_layout_passes=False),
    scratch_types=[
        pltpu.VMEM((2, SCH, 2 * D), jnp.float32),
        pltpu.VMEM((2, SCH, 2 * D), jnp.float32),
        pltpu.VMEM((2, SCH, 2 * D), jnp.float32),
        pltpu.VMEM((SB,), jnp.float32),
        pltpu.VMEM((SB,), jnp.float32),
        pltpu.SemaphoreType.DMA,
        pltpu.SemaphoreType.DMA,
    ],
)
def _score(urows_h, prows_h, nrows_h, pos_h, neg_h,
           ub, pb, nb, posv, negv, sem0, sem1):
    w = lax.axis_index("s") * NC + lax.axis_index("c")
    base = w * SB
    lanes = lax.iota(jnp.int32, L)
    nch = SB // SCH

    def fire(q, sem):
        bb = q % 2
        return (
            pltpu.async_copy(urows_h.at[pl.ds(base + q * SCH, SCH)], ub.at[bb], sem),
            pltpu.async_copy(prows_h.at[pl.ds(base + q * SCH, SCH)], pb.at[bb], sem),
            pltpu.async_copy(nrows_h.at[pl.ds(base + q * SCH, SCH)], nb.at[bb], sem),
        )

    fire(0, sem0)
    for q in range(nch):
        bb = q % 2
        sem = sem0 if bb == 0 else sem1
        nsem = sem1 if bb == 0 else sem0
        if q + 1 < nch:
            fire(q + 1, nsem)
        pltpu.make_async_copy(
            urows_h.at[pl.ds(base + q * SCH, SCH)], ub.at[bb], sem).wait()
        pltpu.make_async_copy(
            prows_h.at[pl.ds(base + q * SCH, SCH)], pb.at[bb], sem).wait()
        pltpu.make_async_copy(
            nrows_h.at[pl.ds(base + q * SCH, SCH)], nb.at[bb], sem).wait()

        def chunk(g, carry):
            r0 = g * L
            pvec = jnp.zeros((L,), jnp.float32)
            nvec = jnp.zeros((L,), jnp.float32)
            for j in range(L):
                r = r0 + j
                tp = jnp.zeros((L,), jnp.float32)
                tn = jnp.zeros((L,), jnp.float32)
                for k in range(D // L):
                    u = ub[bb, r, pl.ds(k * L, L)]
                    tp = tp + u * pb[bb, r, pl.ds(k * L, L)]
                    tn = tn + u * nb[bb, r, pl.ds(k * L, L)]
                pvec = jnp.where(lanes == j, jnp.sum(tp), pvec)
                nvec = jnp.where(lanes == j, jnp.sum(tn), nvec)
            posv[pl.ds(q * SCH + r0, L)] = pvec
            negv[pl.ds(q * SCH + r0, L)] = nvec
            return carry

        lax.fori_loop(0, SCH // L, chunk, 0)

    pltpu.sync_copy(posv, pos_h.at[pl.ds(base, SB)])
    pltpu.sync_copy(negv, neg_h.at[pl.ds(base, SB)])


def kernel(user_ids, pos_ids, neg_ids, user_table, item_table):
    ur, pr, nr = _sweep(user_ids, pos_ids, neg_ids, user_table.T, item_table.T)
    return _score(ur, pr, nr)


# speedup vs baseline: 1.0038x; 1.0038x over previous
import functools

import jax
import jax.numpy as jnp
from jax import lax
from jax.experimental import pallas as pl
from jax.experimental.pallas import tpu as pltpu
from jax.experimental.pallas import tpu_sc as plsc

V = 1000000
D = 64
B = 16384
L = 16
NC, NS = 2, 16
NW = NC * NS
CH = 256
NFULL = 3906
MINI = NFULL * CH
MINIW = V - MINI
NLOC = 123
MINIOWN = 2
CAP = 48
RS = 32
TRASH = B
OUTR = B + 8
IDW = 2048

_mesh = plsc.VectorSubcoreMesh(core_axis_name="c", subcore_axis_name="s")


@functools.partial(
    pl.kernel,
    out_type=(
        jax.ShapeDtypeStruct((OUTR, 2 * D), jnp.float32),
        jax.ShapeDtypeStruct((OUTR, 2 * D), jnp.float32),
        jax.ShapeDtypeStruct((OUTR, 2 * D), jnp.float32),
    ),
    mesh=_mesh,
    compiler_params=pltpu.CompilerParams(needs_layout_passes=False),
    scratch_types=[
        pltpu.VMEM((2, D, CH), jnp.float32),
        pltpu.VMEM((2, D, CH), jnp.float32),
        pltpu.VMEM((D, MINIW), jnp.float32),
        pltpu.VMEM((NLOC * CAP,), jnp.int32),
        pltpu.VMEM((NLOC * CAP,), jnp.int32),
        pltpu.VMEM((NLOC * CAP,), jnp.int32),
        pltpu.VMEM((RS, 2 * D), jnp.float32),
        pltpu.VMEM((RS, 2 * D), jnp.float32),
        pltpu.VMEM((RS, 2 * D), jnp.float32),
        pltpu.VMEM((3, RS), jnp.int32),
        pltpu.VMEM((2, IDW), jnp.int32),
        pltpu.SMEM((3, NLOC), jnp.int32),
        pltpu.SemaphoreType.DMA,
        pltpu.SemaphoreType.DMA,
        pltpu.SemaphoreType.DMA,
        pltpu.SemaphoreType.DMA,
        pltpu.SemaphoreType.DMA,
        pltpu.SemaphoreType.DMA,
    ],
)
def _sweep(uids_h, pids_h, nids_h, utab_h, itab_h,
           urows_h, prows_h, nrows_h,
           ubuf, ibuf, minibuf, ubkt, pbkt, nbkt, rsu, rsp, rsn, slots, idst,
           cur, semu0, semu1, semi0, semi1, semid, semsc):
    w = lax.axis_index("s") * NC + lax.axis_index("c")
    lanes = lax.iota(jnp.int32, L)
    lane0 = lanes == 0
    trash = jnp.full((L,), TRASH, jnp.int32) + (w & 7)

    def zcur(i, c):
        cur[0, i] = 0
        cur[1, i] = 0
        cur[2, i] = 0
        return c

    lax.fori_loop(0, NLOC, zcur, 0)
    for li in range(3):
        for kk in range(RS // L):
            slots[li, pl.ds(kk * L, L)] = trash

    def chunk_start(j):
        return jnp.minimum(w + NW * j, NFULL - 1) * CH

    def fire_chunk(tab_h, buf, j, bb, sem):
        return pltpu.async_copy(
            tab_h.at[:, pl.ds(chunk_start(j), CH)], buf.at[bb], sem)

    def wait_chunk(tab_h, buf, bb, sem):
        pltpu.make_async_copy(
            tab_h.at[:, pl.ds(0, CH)], buf.at[bb], sem).wait()

    fire_chunk(utab_h, ubuf, jnp.int32(0), 0, semu0)
    fire_chunk(itab_h, ibuf, jnp.int32(0), 0, semi0)
    fire_chunk(utab_h, ubuf, jnp.int32(1), 1, semu1)
    fire_chunk(itab_h, ibuf, jnp.int32(1), 1, semi1)

    ids_list = (uids_h, pids_h, nids_h)
    bkts = (ubkt, pbkt, nbkt)
    nwin = B // IDW

    def fire_ids(li, wi):
        return pltpu.async_copy(
            ids_list[li].at[pl.ds(wi * IDW, IDW)], idst.at[(li * nwin + wi) % 2],
            semid)

    fire_ids(0, 0)
    for li in range(3):
        bkt = bkts[li]
        for wi in range(nwin):
            seq = li * nwin + wi
            bb = seq % 2
            if seq + 1 < 3 * nwin:
                nli, nwi = divmod(seq + 1, nwin)
                fire_ids(nli, nwi)
            pltpu.make_async_copy(
                ids_list[li].at[pl.ds(wi * IDW, IDW)], idst.at[bb], semid
            ).wait()

            def group(s, c):
                v = idst[bb, pl.ds(s * L, L)]
                g = lax.shift_right_logical(v, 8)
                own = (g & 31) == w
                o = v - g * CH
                slot = wi * IDW + s * L + lanes
                entry = slot * 1024 + o
                lvec = lax.shift_right_logical(g, 5)

                def cond(m):
                    return jnp.any(m)

                def body(m):
                    j16 = plsc.all_reduce_ffs(m)
                    sel = lanes == j16
                    e = jnp.max(jnp.where(sel, entry, 0))
                    l = jnp.max(jnp.where(sel, lvec, 0))
                    c0 = cur[li, l]

                    @pl.when(c0 < CAP)
                    def _():
                        plsc.store_scatter(
                            bkt, [jnp.full((L,), l * CAP + c0, jnp.int32)],
                            jnp.full((L,), e, jnp.int32), mask=lane0)
                        cur[li, l] = c0 + 1

                    return m & jnp.logical_not(sel)

                lax.while_loop(cond, body, own)
                return c

            lax.fori_loop(0, IDW // L, group, 0)

    cvecs = [lanes + 16 * k for k in range(D // L)]

    def drain(bkt, li, j, cb, rstage, rows_h, rc, cnt):
        def one(m, rc):
            ev = bkt[pl.ds(j * CAP + (m & ~15), L)]
            sel = lanes == (m & 15)
            e = jnp.max(jnp.where(sel, ev, 0))
            o = e & 1023
            slot = lax.shift_right_logical(e, 10)
            r = rc & (RS - 1)
            ov = jnp.full((L,), o, jnp.int32)
            for k in range(D // L):
                vals = plsc.load_gather(cb, [cvecs[k], ov])
                rstage[r, pl.ds(k * L, L)] = vals
            plsc.store_scatter(
                slots, [jnp.full((L,), li, jnp.int32),
                        jnp.full((L,), r, jnp.int32)],
                jnp.full((L,), slot, jnp.int32), mask=lane0)
            rc = rc + 1

            @pl.when(rc & (RS - 1) == 0)
            def _():
                pltpu.async_copy(rstage, rows_h.at[slots.at[li]], semsc).wait()
                for kk in range(RS // L):
                    slots[li, pl.ds(kk * L, L)] = trash

            return rc

        return lax.fori_loop(0, cnt, one, rc)

    def full_cnt(li, j):
        c = cur[li, j]
        return jnp.where((j == NLOC - 1) & (w == MINIOWN), 0, c)

    def mini_cnt(li):
        return jnp.where(w == MINIOWN, cur[li, NLOC - 1], 0)

    def drains(j, ub_bb, ib_bb, rcu, rcp, rcn):
        wait_chunk(utab_h, ubuf, ub_bb, semu0 if ub_bb == 0 else semu1)
        rcu = drain(ubkt, 0, j, ubuf.at[ub_bb], rsu, urows_h, rcu,
                    full_cnt(0, j))
        wait_chunk(itab_h, ibuf, ib_bb, semi0 if ib_bb == 0 else semi1)
        rcp = drain(pbkt, 1, j, ibuf.at[ib_bb], rsp, prows_h, rcp,
                    full_cnt(1, j))
        rcn = drain(nbkt, 2, j, ibuf.at[ib_bb], rsn, nrows_h, rcn,
                    full_cnt(2, j))
        return rcu, rcp, rcn

    rcu = jnp.int32(0)
    rcp = jnp.int32(0)
    rcn = jnp.int32(0)

    def sweep(jj, carry):
        rcu, rcp, rcn = carry
        j0 = 2 * jj

        @pl.when(j0 > 0)
        def _():
            fire_chunk(utab_h, ubuf, j0 + 1, 1, semu1)
            fire_chunk(itab_h, ibuf, j0 + 1, 1, semi1)

        rcu, rcp, rcn = drains(j0, 0, 0, rcu, rcp, rcn)

        @pl.when(j0 + 2 < NLOC)
        def _():
            fire_chunk(utab_h, ubuf, j0 + 2, 0, semu0)
            fire_chunk(itab_h, ibuf, j0 + 2, 0, semi0)

        rcu, rcp, rcn = drains(j0 + 1, 1, 1, rcu, rcp, rcn)
        return rcu, rcp, rcn

    rcu, rcp, rcn = lax.fori_loop(0, NLOC // 2, sweep, (rcu, rcp, rcn))
    rcu, rcp, rcn = drains(NLOC - 1, 0, 0, rcu, rcp, rcn)

    pltpu.sync_copy(utab_h.at[:, pl.ds(MINI, MINIW)], minibuf)
    rcu = drain(ubkt, 0, NLOC - 1, minibuf, rsu, urows_h, rcu, mini_cnt(0))
    pltpu.sync_copy(itab_h.at[:, pl.ds(MINI, MINIW)], minibuf)
    rcp = drain(pbkt, 1, NLOC - 1, minibuf, rsp, prows_h, rcp, mini_cnt(1))
    rcn = drain(nbkt, 2, NLOC - 1, minibuf, rsn, nrows_h, rcn, mini_cnt(2))

    for li, rstage, rows_h, rc in ((0, rsu, urows_h, rcu),
                                   (1, rsp, prows_h, rcp),
                                   (2, rsn, nrows_h, rcn)):
        @pl.when(rc & (RS - 1) != 0)
        def _():
            pltpu.async_copy(rstage, rows_h.at[slots.at[li]], semsc).wait()


SB = B // NW
SCH = 128


@functools.partial(
    pl.kernel,
    out_type=(
        jax.ShapeDtypeStruct((B,), jnp.float32),
        jax.ShapeDtypeStruct((B,), jnp.float32),
    ),
    mesh=_mesh,
    compiler_params=pltpu.CompilerParams(needs_layout_passes=False),
    scratch_types=[
        pltpu.VMEM((2, SCH, 2 * D), jnp.float32),
        pltpu.VMEM((2, SCH, 2 * D), jnp.float32),
        pltpu.VMEM((2, SCH, 2 * D), jnp.float32),
        pltpu.VMEM((SB,), jnp.float32),
        pltpu.VMEM((SB,), jnp.float32),
        pltpu.SemaphoreType.DMA,
        pltpu.SemaphoreType.DMA,
    ],
)
def _score(urows_h, prows_h, nrows_h, pos_h, neg_h,
           ub, pb, nb, posv, negv, sem0, sem1):
    w = lax.axis_index("s") * NC + lax.axis_index("c")
    base = w * SB
    lanes = lax.iota(jnp.int32, L)
    nch = SB // SCH

    def fire(q, sem):
        bb = q % 2
        return (
            pltpu.async_copy(urows_h.at[pl.ds(base + q * SCH, SCH)], ub.at[bb], sem),
            pltpu.async_copy(prows_h.at[pl.ds(base + q * SCH, SCH)], pb.at[bb], sem),
            pltpu.async_copy(nrows_h.at[pl.ds(base + q * SCH, SCH)], nb.at[bb], sem),
        )

    fire(0, sem0)
    for q in range(nch):
        bb = q % 2
        sem = sem0 if bb == 0 else sem1
        nsem = sem1 if bb == 0 else sem0
        if q + 1 < nch:
            fire(q + 1, nsem)
        pltpu.make_async_copy(
            urows_h.at[pl.ds(base + q * SCH, SCH)], ub.at[bb], sem).wait()
        pltpu.make_async_copy(
            prows_h.at[pl.ds(base + q * SCH, SCH)], pb.at[bb], sem).wait()
        pltpu.make_async_copy(
            nrows_h.at[pl.ds(base + q * SCH, SCH)], nb.at[bb], sem).wait()

        def chunk(g, carry):
            r0 = g * L
            pvec = jnp.zeros((L,), jnp.float32)
            nvec = jnp.zeros((L,), jnp.float32)
            for j in range(L):
                r = r0 + j
                tp = jnp.zeros((L,), jnp.float32)
                tn = jnp.zeros((L,), jnp.float32)
                for k in range(D // L):
                    u = ub[bb, r, pl.ds(k * L, L)]
                    tp = tp + u * pb[bb, r, pl.ds(k * L, L)]
                    tn = tn + u * nb[bb, r, pl.ds(k * L, L)]
                pvec = jnp.where(lanes == j, jnp.sum(tp), pvec)
                nvec = jnp.where(lanes == j, jnp.sum(tn), nvec)
            posv[pl.ds(q * SCH + r0, L)] = pvec
            negv[pl.ds(q * SCH + r0, L)] = nvec
            return carry

        lax.fori_loop(0, SCH // L, chunk, 0)

    pltpu.sync_copy(posv, pos_h.at[pl.ds(base, SB)])
    pltpu.sync_copy(negv, neg_h.at[pl.ds(base, SB)])


def kernel(user_ids, pos_ids, neg_ids, user_table, item_table):
    ur, pr, nr = _sweep(user_ids, pos_ids, neg_ids, user_table.T, item_table.T)
    return _score(ur, pr, nr)
